# baseline (speedup 1.0000x reference)
import functools

import jax
import jax.numpy as jnp
from jax import lax
from jax.experimental import pallas as pl
from jax.experimental.pallas import tpu as pltpu
from jax.experimental.pallas import tpu_sc as plsc

_NUM_CORES = 2
_NUM_SUBCORES = 16
_NW = _NUM_CORES * _NUM_SUBCORES
_CB = 32


@functools.lru_cache(maxsize=None)
def _make_gather(batch: int, hist: int, vocab: int, dim: int):
    b_per_w = batch // _NW
    n_chunks = b_per_w // _CB
    assert n_chunks * _CB == b_per_w
    mesh = plsc.VectorSubcoreMesh(core_axis_name="c", subcore_axis_name="s")

    @functools.partial(
        pl.kernel,
        mesh=mesh,
        out_type=jax.ShapeDtypeStruct((batch, hist, dim), jnp.float32),
        compiler_params=pltpu.CompilerParams(use_tc_tiling_on_sc=False),
        scratch_types=[
            pltpu.VMEM((b_per_w, hist), jnp.int32),
            pltpu.VMEM((_CB, hist, dim), jnp.float32),
            pltpu.VMEM((_CB, hist, dim), jnp.float32),
            pltpu.SemaphoreType.DMA,
            pltpu.SemaphoreType.DMA,
            pltpu.SemaphoreType.DMA,
            pltpu.SemaphoreType.DMA,
        ],
    )
    def gather_kernel(x_hbm, table_hbm, out_hbm, idx_v, r0, r1, gs0, gs1, ss0, ss1):
        rows = (r0, r1)
        gsem = (gs0, gs1)
        ssem = (ss0, ss1)
        wid = lax.axis_index("s") * _NUM_CORES + lax.axis_index("c")
        base_b = wid * b_per_w
        pltpu.sync_copy(x_hbm.at[pl.ds(base_b, b_per_w)], idx_v)

        def start_gather(g):
            b = g % 2
            return [
                pltpu.async_copy(
                    table_hbm.at[idx_v.at[g * _CB + j]], rows[b].at[j], gsem[b])
                for j in range(_CB)
            ]

        def start_store(g):
            b = g % 2
            return pltpu.async_copy(
                rows[b], out_hbm.at[pl.ds(base_b + g * _CB, _CB)], ssem[b])

        gh = [None] * n_chunks
        sh = [None] * n_chunks
        gh[0] = start_gather(0)
        for g in range(n_chunks):
            if g + 1 < n_chunks:
                if g >= 1:
                    sh[g - 1].wait()
                gh[g + 1] = start_gather(g + 1)
            for h in gh[g]:
                h.wait()
            sh[g] = start_store(g)
        sh[n_chunks - 1].wait()
        if n_chunks >= 2:
            sh[n_chunks - 2].wait()

    return gather_kernel


_TBLK = 2048


@functools.lru_cache(maxsize=None)
def _make_format_table(vocab: int, dim: int):
    per = 128 // dim
    out_rows = _TBLK // per

    def body(in_ref, out_ref):
        v = in_ref[...].T.reshape(out_rows, per, dim)
        out_ref[...] = jnp.concatenate(
            [v[:, k, :] for k in range(per)], axis=1)

    return pl.pallas_call(
        body,
        grid=(pl.cdiv(vocab, _TBLK),),
        in_specs=[pl.BlockSpec((dim, _TBLK), lambda i: (0, i))],
        out_specs=pl.BlockSpec((out_rows, 128), lambda i: (i, 0)),
        out_shape=jax.ShapeDtypeStruct((vocab * dim // 128, 128), jnp.float32),
    )


_OB = 16


@functools.lru_cache(maxsize=None)
def _make_format_out(batch: int, hist: int, dim: int):
    per = 128 // dim
    in_rows = _OB * hist // per

    def body(in_ref, out_ref):
        parts = jnp.split(in_ref[...], per, axis=1)
        v = jnp.stack(parts, axis=1).reshape(_OB, hist, dim)
        out_ref[...] = v

    return pl.pallas_call(
        body,
        grid=(batch // _OB,),
        in_specs=[pl.BlockSpec((in_rows, 128), lambda i: (i, 0))],
        out_specs=pl.BlockSpec((_OB, hist, dim), lambda i: (i, 0, 0)),
        out_shape=jax.ShapeDtypeStruct((batch, hist, dim), jnp.float32),
    )


def kernel(x, table):
    batch, hist = x.shape
    vocab, dim = table.shape
    tbl_lin = _make_format_table(vocab, dim)(table.T).reshape(vocab, dim)
    raw = _make_gather(batch, hist, vocab, dim)(x.astype(jnp.int32), tbl_lin)
    flat = raw.reshape(batch * hist * dim // 128, 128)
    return _make_format_out(batch, hist, dim)(flat)


# speedup vs baseline: 1.7067x; 1.7067x over previous
import functools

import jax
import jax.numpy as jnp
from jax import lax
from jax.experimental import pallas as pl
from jax.experimental.pallas import tpu as pltpu
from jax.experimental.pallas import tpu_sc as plsc

_NUM_CORES = 2
_NUM_SUBCORES = 16
_NW = _NUM_CORES * _NUM_SUBCORES
_CB = 32


@functools.lru_cache(maxsize=None)
def _make_gather(batch: int, hist: int, vocab: int, dim: int):
    b_per_w = batch // _NW
    n_chunks = b_per_w // _CB
    assert n_chunks * _CB == b_per_w
    mesh = plsc.VectorSubcoreMesh(core_axis_name="c", subcore_axis_name="s")

    @functools.partial(
        pl.kernel,
        mesh=mesh,
        out_type=jax.ShapeDtypeStruct((batch, hist, dim), jnp.float32),
        compiler_params=pltpu.CompilerParams(use_tc_tiling_on_sc=False),
        scratch_types=[
            pltpu.VMEM((b_per_w, hist), jnp.int32),
            pltpu.VMEM((_CB, hist, dim), jnp.float32),
            pltpu.VMEM((_CB, hist, dim), jnp.float32),
            pltpu.SemaphoreType.DMA,
            pltpu.SemaphoreType.DMA,
            pltpu.SemaphoreType.DMA,
            pltpu.SemaphoreType.DMA,
        ],
    )
    def gather_kernel(x_hbm, table_hbm, out_hbm, idx_v, r0, r1, gs0, gs1, ss0, ss1):
        rows = (r0, r1)
        gsem = (gs0, gs1)
        ssem = (ss0, ss1)
        wid = lax.axis_index("s") * _NUM_CORES + lax.axis_index("c")
        base_b = wid * b_per_w
        pltpu.sync_copy(x_hbm.at[pl.ds(base_b, b_per_w)], idx_v)

        def start_gather(g):
            b = g % 2
            return [
                pltpu.async_copy(
                    table_hbm.at[idx_v.at[g * _CB + j]], rows[b].at[j], gsem[b])
                for j in range(_CB)
            ]

        def start_store(g):
            b = g % 2
            return pltpu.async_copy(
                rows[b], out_hbm.at[pl.ds(base_b + g * _CB, _CB)], ssem[b])

        gh = [None] * n_chunks
        sh = [None] * n_chunks
        gh[0] = start_gather(0)
        for g in range(n_chunks):
            if g + 1 < n_chunks:
                if g >= 1:
                    sh[g - 1].wait()
                gh[g + 1] = start_gather(g + 1)
            for h in gh[g]:
                h.wait()
            sh[g] = start_store(g)
        sh[n_chunks - 1].wait()
        if n_chunks >= 2:
            sh[n_chunks - 2].wait()

    return gather_kernel


_TBLK = 2048


@functools.lru_cache(maxsize=None)
def _make_format_table(vocab: int, dim: int):
    per = 128 // dim
    out_rows = _TBLK // per

    def body(in_ref, out_ref):
        v = in_ref[...].T.reshape(out_rows, per, dim)
        out_ref[...] = jnp.concatenate(
            [v[:, k, :] for k in range(per)], axis=1)

    return pl.pallas_call(
        body,
        grid=(pl.cdiv(vocab, _TBLK),),
        in_specs=[pl.BlockSpec((dim, _TBLK), lambda i: (0, i))],
        out_specs=pl.BlockSpec((out_rows, 128), lambda i: (i, 0)),
        out_shape=jax.ShapeDtypeStruct((vocab * dim // 128, 128), jnp.float32),
    )


def kernel(x, table):
    batch, hist = x.shape
    vocab, dim = table.shape
    tbl_lin = _make_format_table(vocab, dim)(table.T).reshape(vocab, dim)
    return _make_gather(batch, hist, vocab, dim)(x.astype(jnp.int32), tbl_lin)


# speedup vs baseline: 1.7239x; 1.0101x over previous
import functools

import jax
import jax.numpy as jnp
from jax import lax
from jax.experimental import pallas as pl
from jax.experimental.pallas import tpu as pltpu
from jax.experimental.pallas import tpu_sc as plsc

_NUM_CORES = 2
_NUM_SUBCORES = 16
_NW = _NUM_CORES * _NUM_SUBCORES
_CB = 32


@functools.lru_cache(maxsize=None)
def _make_gather(batch: int, hist: int, vocab: int, dim: int):
    b_per_w = batch // _NW
    n_chunks = b_per_w // _CB
    assert n_chunks * _CB == b_per_w
    mesh = plsc.VectorSubcoreMesh(core_axis_name="c", subcore_axis_name="s")

    @functools.partial(
        pl.kernel,
        mesh=mesh,
        out_type=jax.ShapeDtypeStruct((batch, hist, dim), jnp.float32),
        compiler_params=pltpu.CompilerParams(use_tc_tiling_on_sc=False),
        scratch_types=[
            pltpu.VMEM((b_per_w, hist), jnp.int32),
            pltpu.VMEM((_CB, hist, dim), jnp.float32),
            pltpu.VMEM((_CB, hist, dim), jnp.float32),
            pltpu.SemaphoreType.DMA,
            pltpu.SemaphoreType.DMA,
            pltpu.SemaphoreType.DMA,
            pltpu.SemaphoreType.DMA,
        ],
    )
    def gather_kernel(x_hbm, table_hbm, out_hbm, idx_v, r0, r1, gs0, gs1, ss0, ss1):
        rows = (r0, r1)
        gsem = (gs0, gs1)
        ssem = (ss0, ss1)
        wid = lax.axis_index("s") * _NUM_CORES + lax.axis_index("c")
        base_b = wid * b_per_w
        pltpu.sync_copy(x_hbm.at[pl.ds(base_b, b_per_w)], idx_v)

        def start_gather(g):
            b = g % 2
            return [
                pltpu.async_copy(
                    table_hbm.at[idx_v.at[g * _CB + j]], rows[b].at[j], gsem[b])
                for j in range(_CB)
            ]

        def start_store(g):
            b = g % 2
            return pltpu.async_copy(
                rows[b], out_hbm.at[pl.ds(base_b + g * _CB, _CB)], ssem[b])

        gh = [None] * n_chunks
        sh = [None] * n_chunks
        gh[0] = start_gather(0)
        for g in range(n_chunks):
            if g + 1 < n_chunks:
                if g >= 1:
                    sh[g - 1].wait()
                gh[g + 1] = start_gather(g + 1)
            for h in gh[g]:
                h.wait()
            sh[g] = start_store(g)
        sh[n_chunks - 1].wait()
        if n_chunks >= 2:
            sh[n_chunks - 2].wait()

    return gather_kernel


_TBLK = 2048


@functools.lru_cache(maxsize=None)
def _make_format_table(vocab: int, dim: int):
    per = 128 // dim
    out_rows = _TBLK // per

    def body(in_ref, out_ref):
        v = in_ref[...].T.reshape(out_rows, per, dim)
        out_ref[...] = jnp.concatenate(
            [v[:, k, :] for k in range(per)], axis=1)

    return pl.pallas_call(
        body,
        grid=(pl.cdiv(vocab, _TBLK),),
        in_specs=[pl.BlockSpec((dim, _TBLK), lambda i: (0, i))],
        out_specs=pl.BlockSpec((out_rows, 128), lambda i: (i, 0)),
        out_shape=jax.ShapeDtypeStruct((vocab * dim // 128, 128), jnp.float32),
    )


def kernel(x, table):
    batch, hist = x.shape
    vocab, dim = table.shape
    tbl_lin = _make_format_table(vocab, dim)(table.T).reshape(vocab, dim)
    xi = x.astype(jnp.int32)
    half = batch // 2
    gather = _make_gather(half, hist, vocab, dim)
    o1 = gather(xi[:half], tbl_lin)
    o2 = gather(xi[half:], tbl_lin)
    return jnp.concatenate([o1, o2], axis=0)


# speedup vs baseline: 1.7944x; 1.0409x over previous
import functools

import jax
import jax.numpy as jnp
from jax import lax
from jax.experimental import pallas as pl
from jax.experimental.pallas import tpu as pltpu
from jax.experimental.pallas import tpu_sc as plsc

_NUM_CORES = 2
_NUM_SUBCORES = 16
_NW = _NUM_CORES * _NUM_SUBCORES
_CB = 32


@functools.lru_cache(maxsize=None)
def _make_gather(batch: int, hist: int, vocab: int, dim: int):
    b_per_w = batch // _NW
    n_chunks = b_per_w // _CB
    assert n_chunks * _CB == b_per_w
    mesh = plsc.VectorSubcoreMesh(core_axis_name="c", subcore_axis_name="s")

    @functools.partial(
        pl.kernel,
        mesh=mesh,
        out_type=jax.ShapeDtypeStruct((batch, hist, dim), jnp.float32),
        compiler_params=pltpu.CompilerParams(use_tc_tiling_on_sc=False),
        scratch_types=[
            pltpu.VMEM((b_per_w, hist), jnp.int32),
            pltpu.VMEM((_CB, hist, dim), jnp.float32),
            pltpu.VMEM((_CB, hist, dim), jnp.float32),
            pltpu.SemaphoreType.DMA,
            pltpu.SemaphoreType.DMA,
            pltpu.SemaphoreType.DMA,
            pltpu.SemaphoreType.DMA,
        ],
    )
    def gather_kernel(x_hbm, table_hbm, out_hbm, idx_v, r0, r1, gs0, gs1, ss0, ss1):
        rows = (r0, r1)
        gsem = (gs0, gs1)
        ssem = (ss0, ss1)
        wid = lax.axis_index("s") * _NUM_CORES + lax.axis_index("c")
        base_b = wid * b_per_w
        pltpu.sync_copy(x_hbm.at[pl.ds(base_b, b_per_w)], idx_v)

        def start_gather(g):
            b = g % 2
            return [
                pltpu.async_copy(
                    table_hbm.at[idx_v.at[g * _CB + j]], rows[b].at[j], gsem[b])
                for j in range(_CB)
            ]

        def start_store(g):
            b = g % 2
            return pltpu.async_copy(
                rows[b], out_hbm.at[pl.ds(base_b + g * _CB, _CB)], ssem[b])

        gh = [None] * n_chunks
        sh = [None] * n_chunks
        gh[0] = start_gather(0)
        for g in range(n_chunks):
            if g + 1 < n_chunks:
                if g >= 1:
                    sh[g - 1].wait()
                gh[g + 1] = start_gather(g + 1)
            for h in gh[g]:
                h.wait()
            sh[g] = start_store(g)
        sh[n_chunks - 1].wait()
        if n_chunks >= 2:
            sh[n_chunks - 2].wait()

    return gather_kernel


_TBLK = 2048


@functools.lru_cache(maxsize=None)
def _make_format_table(vocab: int, dim: int):
    per = 128 // dim
    out_rows = _TBLK // per

    def body(in_ref, out_ref):
        v = in_ref[...].T.reshape(out_rows, per, dim)
        out_ref[...] = jnp.concatenate(
            [v[:, k, :] for k in range(per)], axis=1)

    return pl.pallas_call(
        body,
        grid=(pl.cdiv(vocab, _TBLK),),
        in_specs=[pl.BlockSpec((dim, _TBLK), lambda i: (0, i))],
        out_specs=pl.BlockSpec((out_rows, 128), lambda i: (i, 0)),
        out_shape=jax.ShapeDtypeStruct((vocab * dim // 128, 128), jnp.float32),
    )


def kernel(x, table):
    batch, hist = x.shape
    vocab, dim = table.shape
    tbl_lin = _make_format_table(vocab, dim)(table.T).reshape(vocab, dim)
    xi = x.astype(jnp.int32)
    nsplit = 4
    part = batch // nsplit
    gather = _make_gather(part, hist, vocab, dim)
    outs = [gather(xi[i * part:(i + 1) * part], tbl_lin) for i in range(nsplit)]
    return jnp.concatenate(outs, axis=0)
